# Initial kernel scaffold; baseline (speedup 1.0000x reference)
import jax
import jax.numpy as jnp
from jax.experimental import pallas as pl


def kernel(x, edge_index, edge_attr, W_node, b_node, W_edge, b_edge, W1_0, b1_0, W2_0, b2_0, g_0, be_0, W1_1, b1_1, W2_1, b2_1, g_1, be_1, W_out, b_out):
    raise NotImplementedError("write your pallas kernel here")


# speedup vs baseline: 3.6832x; 3.6832x over previous
import functools

import jax
import jax.numpy as jnp
from jax import lax
from jax.experimental import pallas as pl
from jax.experimental.pallas import tpu as pltpu
from jax.experimental.pallas import tpu_sc as plsc

N = 10000
E = 320000
D = 128
NC = 2
NS = 16
NW = NC * NS
EPW = E // NW
CHUNK = 128
NFULL = EPW // CHUNK
TAIL = EPW - NFULL * CHUNK
NPAD = 10240
RPT = NPAD // NS
LANES = 16


def _make_sc_edge_aggr():
    mesh = plsc.VectorSubcoreMesh(core_axis_name="c", subcore_axis_name="s")

    @functools.partial(
        pl.kernel,
        out_type=jax.ShapeDtypeStruct((NC, NPAD, D), jnp.float32),
        mesh=mesh,
        compiler_params=pltpu.CompilerParams(use_tc_tiling_on_sc=False),
        scratch_types=[
            pltpu.VMEM_SHARED((NPAD, D), jnp.float32),
            pltpu.VMEM((CHUNK,), jnp.int32),
            pltpu.VMEM((CHUNK,), jnp.int32),
            pltpu.VMEM((TAIL,), jnp.int32),
            pltpu.VMEM((TAIL,), jnp.int32),
            pltpu.VMEM((CHUNK, D), jnp.float32),
            pltpu.VMEM((CHUNK, D), jnp.float32),
            pltpu.VMEM((TAIL, D), jnp.float32),
            pltpu.VMEM((TAIL, D), jnp.float32),
            pltpu.SemaphoreType.DMA,
        ],
    )
    def sc_edge_aggr(h_hbm, e_hbm, src_hbm, dst_hbm, out_hbm,
                     aggr, sidx, didx, sidx_t, didx_t,
                     hbuf, ebuf, hbuf_t, ebuf_t, sem):
        cid = lax.axis_index("c")
        sid = lax.axis_index("s")
        wid = cid * NS + sid

        def zrow(r, carry):
            for k in range(D // LANES):
                hbuf[r, pl.ds(k * LANES, LANES)] = jnp.zeros((LANES,), jnp.float32)
            return carry
        lax.fori_loop(0, CHUNK, zrow, 0)
        for c in range(RPT // CHUNK):
            pltpu.sync_copy(hbuf, aggr.at[pl.ds(sid * RPT + c * CHUNK, CHUNK), :])
        plsc.subcore_barrier()

        base0 = wid * EPW

        def do_chunk(base, size, si, di, hb, eb):
            pltpu.sync_copy(src_hbm.at[pl.ds(base, size)], si)
            gat = pltpu.async_copy(h_hbm.at[si], hb, sem)
            pltpu.sync_copy(dst_hbm.at[pl.ds(base, size)], di)
            pltpu.sync_copy(e_hbm.at[pl.ds(base, size), :], eb)
            gat.wait()

            def row(r, carry):
                for k in range(D // LANES):
                    sl = pl.ds(k * LANES, LANES)
                    hb[r, sl] = jnp.maximum(hb[r, sl] + eb[r, sl], 0.0)
                return carry
            lax.fori_loop(0, size, row, 0)
            pltpu.sync_copy(hb, aggr.at[di], add=True)

        def chunk_loop(c, carry):
            do_chunk(base0 + c * CHUNK, CHUNK, sidx, didx, hbuf, ebuf)
            return carry
        lax.fori_loop(0, NFULL, chunk_loop, 0)
        do_chunk(base0 + NFULL * CHUNK, TAIL, sidx_t, didx_t, hbuf_t, ebuf_t)

        plsc.subcore_barrier()
        for c in range(RPT // CHUNK):
            r0 = sid * RPT + c * CHUNK
            pltpu.sync_copy(aggr.at[pl.ds(r0, CHUNK), :], hbuf)
            pltpu.sync_copy(hbuf, out_hbm.at[cid, pl.ds(r0, CHUNK), :])

    return sc_edge_aggr


def _matmul_bias(x, W, b, block):
    M, K = x.shape
    _, Dout = W.shape

    def body(x_ref, w_ref, b_ref, o_ref):
        o_ref[...] = (
            jnp.dot(x_ref[...], w_ref[...], preferred_element_type=jnp.float32)
            + b_ref[...]
        )

    return pl.pallas_call(
        body,
        grid=(M // block,),
        in_specs=[
            pl.BlockSpec((block, K), lambda i: (i, 0)),
            pl.BlockSpec((K, Dout), lambda i: (0, 0)),
            pl.BlockSpec((1, Dout), lambda i: (0, 0)),
        ],
        out_specs=pl.BlockSpec((block, Dout), lambda i: (i, 0)),
        out_shape=jax.ShapeDtypeStruct((M, Dout), jnp.float32),
    )(x, W, b.reshape(1, Dout))


def _post_layer(h, a0, a1, W1, b1, W2, b2, g, be, Wo=None, bo=None, block=2000):
    with_head = Wo is not None

    def body(h_ref, a0_ref, a1_ref, w1, b1r, w2, b2r, gr, ber, *rest):
        z = h_ref[...] + a0_ref[...] + a1_ref[...]
        t = jnp.maximum(
            jnp.dot(z, w1[...], preferred_element_type=jnp.float32) + b1r[...], 0.0)
        hn = jnp.dot(t, w2[...], preferred_element_type=jnp.float32) + b2r[...]
        mu = jnp.mean(hn, axis=-1, keepdims=True)
        var = jnp.mean((hn - mu) ** 2, axis=-1, keepdims=True)
        y = (hn - mu) * lax.rsqrt(var + 1e-5) * gr[...] + ber[...]
        y = jnp.maximum(y, 0.0)
        if with_head:
            wo, bor, o_ref = rest
            o_ref[...] = (
                jnp.dot(y, wo[...], preferred_element_type=jnp.float32) + bor[...])
        else:
            rest[0][...] = y

    full = lambda shape: pl.BlockSpec(shape, lambda i: tuple(0 for _ in shape))
    in_specs = [
        pl.BlockSpec((block, D), lambda i: (i, 0)),
        pl.BlockSpec((block, D), lambda i: (i, 0)),
        pl.BlockSpec((block, D), lambda i: (i, 0)),
        full((D, D)), full((1, D)), full((D, D)), full((1, D)),
        full((1, D)), full((1, D)),
    ]
    args = [h, a0, a1, W1, b1.reshape(1, D), W2, b2.reshape(1, D),
            g.reshape(1, D), be.reshape(1, D)]
    if with_head:
        in_specs += [full((D, D)), full((1, D))]
        args += [Wo, bo.reshape(1, D)]

    return pl.pallas_call(
        body,
        grid=(N // block,),
        in_specs=in_specs,
        out_specs=pl.BlockSpec((block, D), lambda i: (i, 0)),
        out_shape=jax.ShapeDtypeStruct((N, D), jnp.float32),
    )(*args)


def kernel(x, edge_index, edge_attr, W_node, b_node, W_edge, b_edge,
           W1_0, b1_0, W2_0, b2_0, g_0, be_0,
           W1_1, b1_1, W2_1, b2_1, g_1, be_1,
           W_out, b_out):
    src = edge_index[0]
    dst = edge_index[1]

    h = _matmul_bias(x, W_node, b_node, 2000)
    e = _matmul_bias(edge_attr, W_edge, b_edge, 8000)

    sc_edge_aggr = _make_sc_edge_aggr()

    a = sc_edge_aggr(h, e, src, dst)
    h = _post_layer(h, a[0, :N], a[1, :N], W1_0, b1_0, W2_0, b2_0, g_0, be_0)
    a = sc_edge_aggr(h, e, src, dst)
    out = _post_layer(h, a[0, :N], a[1, :N], W1_1, b1_1, W2_1, b2_1, g_1, be_1,
                      W_out, b_out)
    return out


# speedup vs baseline: 5.0672x; 1.3757x over previous
import functools

import jax
import jax.numpy as jnp
from jax import lax
from jax.experimental import pallas as pl
from jax.experimental.pallas import tpu as pltpu
from jax.experimental.pallas import tpu_sc as plsc

N = 10000
E = 320000
D = 128
NC = 2
NS = 16
NW = NC * NS
CHUNK = 80
CPT = E // (NW * CHUNK)
EROWS = E // CHUNK
NPAD = 10240
RPT = NPAD // NS
ZCH = RPT // CHUNK
LANES = 16


def _make_sc_edge_aggr():
    mesh = plsc.VectorSubcoreMesh(core_axis_name="c", subcore_axis_name="s")

    @functools.partial(
        pl.kernel,
        out_type=jax.ShapeDtypeStruct((NC, NPAD, D), jnp.float32),
        mesh=mesh,
        compiler_params=pltpu.CompilerParams(use_tc_tiling_on_sc=False),
        scratch_types=[
            pltpu.VMEM_SHARED((NPAD, D), jnp.float32),
            pltpu.VMEM((2, CHUNK), jnp.int32),
            pltpu.VMEM((2, CHUNK), jnp.int32),
            pltpu.VMEM((CHUNK, D), jnp.float32),
            pltpu.VMEM((CHUNK, D), jnp.float32),
            pltpu.VMEM((CHUNK, D), jnp.float32),
            pltpu.VMEM((CHUNK, D), jnp.float32),
            pltpu.SemaphoreType.DMA,
            pltpu.SemaphoreType.DMA,
            pltpu.SemaphoreType.DMA,
            pltpu.SemaphoreType.DMA,
        ],
    )
    def sc_edge_aggr(h_hbm, e_hbm, ei_hbm, out_hbm,
                     aggr, idxA, idxB, hA, hB, eA, eB,
                     isA, isB, gsA, gsB):
        cid = lax.axis_index("c")
        sid = lax.axis_index("s")
        wid = cid * NS + sid
        row0 = wid * CPT

        idx = (idxA, idxB)
        hb = (hA, hB)
        eb = (eA, eB)
        isem = (isA, isB)
        gsem = (gsA, gsB)

        def zrow(r, carry):
            for k in range(D // LANES):
                hA[r, pl.ds(k * LANES, LANES)] = jnp.zeros((LANES,), jnp.float32)
            return carry
        lax.fori_loop(0, CHUNK, zrow, 0)
        for z in range(ZCH):
            pltpu.sync_copy(hA, aggr.at[pl.ds(sid * RPT + z * CHUNK, CHUNK), :])
        plsc.subcore_barrier()

        def load_idx(c, b):
            return pltpu.async_copy(ei_hbm.at[:, row0 + c, :], idx[b], isem[b])

        def issue_gather(c, b):
            pltpu.async_copy(h_hbm.at[idx[b].at[0]], hb[b], gsem[b])
            pltpu.async_copy(e_hbm.at[pl.ds((row0 + c) * CHUNK, CHUNK), :],
                             eb[b], gsem[b])

        def wait_idx(b):
            pltpu.make_async_copy(ei_hbm.at[:, 0, :], idx[b], isem[b]).wait()

        def wait_gather(b):
            pltpu.make_async_copy(h_hbm.at[idx[b].at[0]], hb[b], gsem[b]).wait()
            pltpu.make_async_copy(e_hbm.at[pl.ds(0, CHUNK), :], eb[b],
                                  gsem[b]).wait()

        def compute_scatter(b):
            def rowfn(r, carry):
                for k in range(D // LANES):
                    sl = pl.ds(k * LANES, LANES)
                    hb[b][r, sl] = jnp.maximum(hb[b][r, sl] + eb[b][r, sl], 0.0)
                return carry
            lax.fori_loop(0, CHUNK, rowfn, 0)
            pltpu.sync_copy(hb[b], aggr.at[idx[b].at[1]], add=True)

        load_idx(0, 0).wait()
        load_idx(1, 1)
        issue_gather(0, 0)

        def step(c, cur, nxt):
            wait_idx(nxt)
            issue_gather(c + 1, nxt)
            wait_gather(cur)
            compute_scatter(cur)

            @pl.when(c + 2 <= CPT - 1)
            def _():
                load_idx(c + 2, cur)

        def pair(i, carry):
            c = 2 * i
            step(c, 0, 1)
            step(c + 1, 1, 0)
            return carry
        lax.fori_loop(0, (CPT - 1) // 2, pair, 0)

        wait_gather(0)
        compute_scatter(0)

        plsc.subcore_barrier()
        for z in range(ZCH):
            r0 = sid * RPT + z * CHUNK
            pltpu.sync_copy(aggr.at[pl.ds(r0, CHUNK), :], hA)
            pltpu.sync_copy(hA, out_hbm.at[cid, pl.ds(r0, CHUNK), :])

    return sc_edge_aggr


def _matmul_bias(x, W, b, block):
    M, K = x.shape
    _, Dout = W.shape

    def body(x_ref, w_ref, b_ref, o_ref):
        o_ref[...] = (
            jnp.dot(x_ref[...], w_ref[...], preferred_element_type=jnp.float32)
            + b_ref[...]
        )

    return pl.pallas_call(
        body,
        grid=(M // block,),
        in_specs=[
            pl.BlockSpec((block, K), lambda i: (i, 0)),
            pl.BlockSpec((K, Dout), lambda i: (0, 0)),
            pl.BlockSpec((1, Dout), lambda i: (0, 0)),
        ],
        out_specs=pl.BlockSpec((block, Dout), lambda i: (i, 0)),
        out_shape=jax.ShapeDtypeStruct((M, Dout), jnp.float32),
    )(x, W, b.reshape(1, Dout))


def _post_layer(h, a0, a1, W1, b1, W2, b2, g, be, Wo=None, bo=None, block=2000):
    with_head = Wo is not None

    def body(h_ref, a0_ref, a1_ref, w1, b1r, w2, b2r, gr, ber, *rest):
        z = h_ref[...] + a0_ref[...] + a1_ref[...]
        t = jnp.maximum(
            jnp.dot(z, w1[...], preferred_element_type=jnp.float32) + b1r[...], 0.0)
        hn = jnp.dot(t, w2[...], preferred_element_type=jnp.float32) + b2r[...]
        mu = jnp.mean(hn, axis=-1, keepdims=True)
        var = jnp.mean((hn - mu) ** 2, axis=-1, keepdims=True)
        y = (hn - mu) * lax.rsqrt(var + 1e-5) * gr[...] + ber[...]
        y = jnp.maximum(y, 0.0)
        if with_head:
            wo, bor, o_ref = rest
            o_ref[...] = (
                jnp.dot(y, wo[...], preferred_element_type=jnp.float32) + bor[...])
        else:
            rest[0][...] = y

    full = lambda shape: pl.BlockSpec(shape, lambda i: tuple(0 for _ in shape))
    in_specs = [
        pl.BlockSpec((block, D), lambda i: (i, 0)),
        pl.BlockSpec((block, D), lambda i: (i, 0)),
        pl.BlockSpec((block, D), lambda i: (i, 0)),
        full((D, D)), full((1, D)), full((D, D)), full((1, D)),
        full((1, D)), full((1, D)),
    ]
    args = [h, a0, a1, W1, b1.reshape(1, D), W2, b2.reshape(1, D),
            g.reshape(1, D), be.reshape(1, D)]
    if with_head:
        in_specs += [full((D, D)), full((1, D))]
        args += [Wo, bo.reshape(1, D)]

    return pl.pallas_call(
        body,
        grid=(N // block,),
        in_specs=in_specs,
        out_specs=pl.BlockSpec((block, D), lambda i: (i, 0)),
        out_shape=jax.ShapeDtypeStruct((N, D), jnp.float32),
    )(*args)


def kernel(x, edge_index, edge_attr, W_node, b_node, W_edge, b_edge,
           W1_0, b1_0, W2_0, b2_0, g_0, be_0,
           W1_1, b1_1, W2_1, b2_1, g_1, be_1,
           W_out, b_out):
    ei = edge_index.reshape(2, EROWS, CHUNK)

    h = _matmul_bias(x, W_node, b_node, 2000)
    e = _matmul_bias(edge_attr, W_edge, b_edge, 8000)

    sc_edge_aggr = _make_sc_edge_aggr()

    a = sc_edge_aggr(h, e, ei)
    h = _post_layer(h, a[0, :N], a[1, :N], W1_0, b1_0, W2_0, b2_0, g_0, be_0)
    a = sc_edge_aggr(h, e, ei)
    out = _post_layer(h, a[0, :N], a[1, :N], W1_1, b1_1, W2_1, b2_1, g_1, be_1,
                      W_out, b_out)
    return out
